# baseline (speedup 1.0000x reference)
import functools

import jax
import jax.numpy as jnp
from jax import lax
from jax.experimental import pallas as pl
from jax.experimental.pallas import tpu as pltpu
from jax.experimental.pallas import tpu_sc as plsc

_NCLS = 1000
_NS = 5
_HID = 512
_XD = 64
_B = 1024
_POS_INF = 1e6
_DN = (((1,), (0,)), ((), ()))
_HI = jax.lax.Precision.HIGHEST

_SC_NC = 2
_SC_NS = 16
_SC_NW = _SC_NC * _SC_NS
_ROWS_PER_W = (2 * _B) // _SC_NW


def _stage1_kernel(x_ref, xt_ref, z_ref, attrs_ref, w1_ref, b1_ref, w2_ref,
                   b2_ref, cent_ref, idx_ref):
    w1 = w1_ref[:]
    w2 = w2_ref[:]
    b1 = b1_ref[:]
    attrs = attrs_ref[:]
    xfsum = jnp.zeros((_NCLS, _XD), jnp.float32)
    for s in range(_NS):
        z_s = z_ref[pl.ds(s * _NCLS, _NCLS), :]
        g_s = jnp.concatenate([z_s, attrs], axis=1)
        h_s = jnp.maximum(jax.lax.dot_general(g_s, w1, _DN) + b1, 0.0)
        xfsum = xfsum + jax.lax.dot_general(h_s, w2, _DN)
    cent = xfsum * jnp.float32(1.0 / _NS) + b2_ref[:]
    cent_ref[:] = jnp.concatenate(
        [cent, jnp.zeros((_NCLS, _XD), jnp.float32)], axis=1)

    cn = jnp.sum(cent * cent, axis=1, keepdims=True)
    cx = jax.lax.dot_general(cent, xt_ref[:], _DN, precision=_HI)
    s_hat = cn - (cx + cx)

    big = jnp.int32(2 ** 30)
    row = jax.lax.broadcasted_iota(jnp.int32, s_hat.shape, 0)
    v1 = jnp.min(s_hat, axis=0, keepdims=True)
    i1 = jnp.min(jnp.where(s_hat == v1, row, big), axis=0, keepdims=True)
    masked = jnp.where(row == i1, jnp.float32(jnp.inf), s_hat)
    v2 = jnp.min(masked, axis=0, keepdims=True)
    i2 = jnp.min(jnp.where(masked == v2, row, big), axis=0, keepdims=True)
    idx_ref[0:1, :] = i1
    idx_ref[1:2, :] = i2


_sc_mesh = plsc.VectorSubcoreMesh(core_axis_name="c", subcore_axis_name="s")


@functools.partial(
    pl.kernel,
    mesh=_sc_mesh,
    out_type=jax.ShapeDtypeStruct((2 * _B, 2 * _XD), jnp.float32),
    scratch_types=[
        pltpu.VMEM((_ROWS_PER_W,), jnp.int32),
        pltpu.VMEM((_ROWS_PER_W, 2 * _XD), jnp.float32),
        pltpu.SemaphoreType.DMA,
    ],
)
def _sc_gather(cent_hbm, idx_hbm, out_hbm, idx_v, rows_v, sem):
    wid = lax.axis_index("s") * _SC_NC + lax.axis_index("c")
    base = wid * _ROWS_PER_W
    pltpu.sync_copy(idx_hbm.at[pl.ds(base, _ROWS_PER_W)], idx_v)
    pltpu.async_copy(cent_hbm.at[idx_v], rows_v, sem).wait()
    pltpu.sync_copy(rows_v, out_hbm.at[pl.ds(base, _ROWS_PER_W)])


def _stage3_kernel(x_ref, c12_ref, idx_ref, out_ref):
    x = x_ref[:]
    c1 = c12_ref[pl.ds(0, _B), pl.ds(0, _XD)]
    c2 = c12_ref[pl.ds(_B, _B), pl.ds(0, _XD)]
    diff1 = c1 - x
    d1 = jnp.sum(diff1 * diff1, axis=-1, keepdims=True)
    diff2 = c2 - x
    d2 = jnp.sum(diff2 * diff2, axis=-1, keepdims=True)
    i1c = jnp.transpose(idx_ref[0:1, :].astype(jnp.float32)).astype(jnp.int32)
    i2c = jnp.transpose(idx_ref[1:2, :].astype(jnp.float32)).astype(jnp.int32)
    w = jnp.where(d1 < d2, i1c,
                  jnp.where(d2 < d1, i2c, jnp.minimum(i1c, i2c)))
    col = jax.lax.broadcasted_iota(jnp.int32, (_B, _NCLS), 1)
    out_ref[:] = jnp.where(col == w, jnp.float32(_POS_INF), jnp.float32(0.0))


def kernel(x, attrs, z, G_W1, G_b1, G_W2, G_b2):
    cent, idx2 = pl.pallas_call(
        _stage1_kernel,
        out_shape=(
            jax.ShapeDtypeStruct((_NCLS, 2 * _XD), jnp.float32),
            jax.ShapeDtypeStruct((2, _B), jnp.int32),
        ),
        compiler_params=pltpu.CompilerParams(vmem_limit_bytes=64 * 1024 * 1024),
    )(x, x.T, z, attrs, G_W1, G_b1.reshape(1, _HID), G_W2, G_b2.reshape(1, _XD))
    c12 = _sc_gather(cent, idx2.reshape(2 * _B))
    return pl.pallas_call(
        _stage3_kernel,
        out_shape=jax.ShapeDtypeStruct((_B, _NCLS), jnp.float32),
        compiler_params=pltpu.CompilerParams(vmem_limit_bytes=64 * 1024 * 1024),
    )(x, c12, idx2)


# speedup vs baseline: 1.4229x; 1.4229x over previous
import jax
import jax.numpy as jnp
from jax.experimental import pallas as pl
from jax.experimental.pallas import tpu as pltpu

_NCLS = 1000
_NS = 5
_HID = 512
_XD = 64
_B = 1024
_POS_INF = 1e6
_DN = (((1,), (0,)), ((), ()))
_HI = jax.lax.Precision.HIGHEST


def _fgc_kernel(x_ref, xt_ref, z_ref, attrs_ref, w1_ref, b1_ref, w2_ref, b2_ref,
                out_ref):
    w1 = w1_ref[:]
    w2 = w2_ref[:]
    b1 = b1_ref[:]
    attrs = attrs_ref[:]
    xfsum = jnp.zeros((_NCLS, _XD), jnp.float32)
    for s in range(_NS):
        z_s = z_ref[pl.ds(s * _NCLS, _NCLS), :]
        g_s = jnp.concatenate([z_s, attrs], axis=1)
        h_s = jnp.maximum(jax.lax.dot_general(g_s, w1, _DN) + b1, 0.0)
        xfsum = xfsum + jax.lax.dot_general(h_s, w2, _DN)
    cent = xfsum * jnp.float32(1.0 / _NS) + b2_ref[:]

    cn = jnp.sum(cent * cent, axis=1, keepdims=True)
    cx = jax.lax.dot_general(cent, xt_ref[:], _DN, precision=_HI)
    s_hat = cn - (cx + cx)

    big = jnp.int32(2 ** 30)
    row = jax.lax.broadcasted_iota(jnp.int32, s_hat.shape, 0)
    v1 = jnp.min(s_hat, axis=0, keepdims=True)
    i1 = jnp.min(jnp.where(s_hat == v1, row, big), axis=0, keepdims=True)
    masked = jnp.where(row == i1, jnp.float32(jnp.inf), s_hat)
    v2 = jnp.min(masked, axis=0, keepdims=True)
    i2 = jnp.min(jnp.where(masked == v2, row, big), axis=0, keepdims=True)

    i1c = jnp.transpose(i1.astype(jnp.float32)).astype(jnp.int32)
    i2c = jnp.transpose(i2.astype(jnp.float32)).astype(jnp.int32)

    x = x_ref[:]
    col = jax.lax.broadcasted_iota(jnp.int32, (_B, _NCLS), 1)
    oh1 = (col == i1c).astype(jnp.float32)
    c1 = jax.lax.dot_general(oh1, cent, _DN, precision=_HI)
    oh2 = (col == i2c).astype(jnp.float32)
    c2 = jax.lax.dot_general(oh2, cent, _DN, precision=_HI)
    diff1 = c1 - x
    d1 = jnp.sum(diff1 * diff1, axis=-1, keepdims=True)
    diff2 = c2 - x
    d2 = jnp.sum(diff2 * diff2, axis=-1, keepdims=True)
    w = jnp.where(d1 < d2, i1c,
                  jnp.where(d2 < d1, i2c, jnp.minimum(i1c, i2c)))
    out_ref[:] = jnp.where(col == w, jnp.float32(_POS_INF), jnp.float32(0.0))


def kernel(x, attrs, z, G_W1, G_b1, G_W2, G_b2):
    return pl.pallas_call(
        _fgc_kernel,
        out_shape=jax.ShapeDtypeStruct((x.shape[0], _NCLS), jnp.float32),
        compiler_params=pltpu.CompilerParams(vmem_limit_bytes=64 * 1024 * 1024),
    )(x, x.T, z, attrs, G_W1, G_b1.reshape(1, _HID), G_W2, G_b2.reshape(1, _XD))


# speedup vs baseline: 1.4275x; 1.0032x over previous
import jax
import jax.numpy as jnp
from jax.experimental import pallas as pl
from jax.experimental.pallas import tpu as pltpu

_NCLS = 1000
_NS = 5
_HID = 512
_XD = 64
_B = 1024
_POS_INF = 1e6
_DN = (((1,), (0,)), ((), ()))
_HI = jax.lax.Precision.HIGHEST


def _fgc_kernel(x_ref, z_ref, attrs_ref, w1_ref, b1_ref, w2_ref, b2_ref,
                out_ref):
    w1 = w1_ref[:]
    w2 = w2_ref[:]
    b1 = b1_ref[:]
    attrs = attrs_ref[:]
    xfsum = jnp.zeros((_NCLS, _XD), jnp.float32)
    for s in range(_NS):
        z_s = z_ref[pl.ds(s * _NCLS, _NCLS), :]
        g_s = jnp.concatenate([z_s, attrs], axis=1)
        h_s = jnp.maximum(jax.lax.dot_general(g_s, w1, _DN) + b1, 0.0)
        xfsum = xfsum + jax.lax.dot_general(h_s, w2, _DN)
    cent = xfsum * jnp.float32(1.0 / _NS) + b2_ref[:]

    cn = jnp.sum(cent * cent, axis=1, keepdims=True)
    cx = jax.lax.dot_general(cent, x_ref[:], (((1,), (1,)), ((), ())),
                             precision=_HI)
    s_hat = cn - (cx + cx)

    big = jnp.int32(2 ** 30)
    row = jax.lax.broadcasted_iota(jnp.int32, s_hat.shape, 0)
    v1 = jnp.min(s_hat, axis=0, keepdims=True)
    i1 = jnp.min(jnp.where(s_hat == v1, row, big), axis=0, keepdims=True)
    masked = jnp.where(row == i1, jnp.float32(jnp.inf), s_hat)
    v2 = jnp.min(masked, axis=0, keepdims=True)
    i2 = jnp.min(jnp.where(masked == v2, row, big), axis=0, keepdims=True)

    i1c = jnp.transpose(i1.astype(jnp.float32)).astype(jnp.int32)
    i2c = jnp.transpose(i2.astype(jnp.float32)).astype(jnp.int32)

    x = x_ref[:]
    col = jax.lax.broadcasted_iota(jnp.int32, (_B, _NCLS), 1)
    oh1 = (col == i1c).astype(jnp.float32)
    c1 = jax.lax.dot_general(oh1, cent, _DN, precision=_HI)
    oh2 = (col == i2c).astype(jnp.float32)
    c2 = jax.lax.dot_general(oh2, cent, _DN, precision=_HI)
    diff1 = c1 - x
    d1 = jnp.sum(diff1 * diff1, axis=-1, keepdims=True)
    diff2 = c2 - x
    d2 = jnp.sum(diff2 * diff2, axis=-1, keepdims=True)
    w = jnp.where(d1 < d2, i1c,
                  jnp.where(d2 < d1, i2c, jnp.minimum(i1c, i2c)))
    out_ref[:] = jnp.where(col == w, jnp.float32(_POS_INF), jnp.float32(0.0))


def kernel(x, attrs, z, G_W1, G_b1, G_W2, G_b2):
    return pl.pallas_call(
        _fgc_kernel,
        out_shape=jax.ShapeDtypeStruct((x.shape[0], _NCLS), jnp.float32),
        compiler_params=pltpu.CompilerParams(vmem_limit_bytes=64 * 1024 * 1024),
    )(x, z, attrs, G_W1, G_b1.reshape(1, _HID), G_W2, G_b2.reshape(1, _XD))


# speedup vs baseline: 2.1097x; 1.4779x over previous
import jax
import jax.numpy as jnp
from jax.experimental import pallas as pl
from jax.experimental.pallas import tpu as pltpu

_NCLS = 1000
_NS = 5
_HID = 512
_XD = 64
_B = 1024
_POS_INF = 1e6
_HI = jax.lax.Precision.HIGHEST


def _fgc_kernel(xt_ref, zt_ref, attrst_ref, w1_ref, b1_ref, w2t_ref, b2_ref,
                outt_ref):
    w1 = w1_ref[:]
    w2t = w2t_ref[:]
    b1 = b1_ref[:]
    attrst = attrst_ref[:]
    xfsum = jnp.zeros((_NCLS, _XD), jnp.float32)
    for s in range(_NS):
        z_st = zt_ref[:, pl.ds(s * _NCLS, _NCLS)]
        g_st = jnp.concatenate([z_st, attrst], axis=0)
        h_s = jnp.maximum(
            jax.lax.dot_general(g_st, w1, (((0,), (0,)), ((), ()))) + b1, 0.0)
        xfsum = xfsum + jax.lax.dot_general(
            h_s, w2t, (((1,), (1,)), ((), ())))
    cent = xfsum * jnp.float32(1.0 / _NS) + b2_ref[:]

    cn = jnp.sum(cent * cent, axis=1, keepdims=True)
    cx = jax.lax.dot_general(cent, xt_ref[:], (((1,), (0,)), ((), ())),
                             precision=_HI)
    s_hat = cn - (cx + cx)

    big = jnp.int32(2 ** 30)
    row = jax.lax.broadcasted_iota(jnp.int32, s_hat.shape, 0)
    v1 = jnp.min(s_hat, axis=0, keepdims=True)
    i1 = jnp.min(jnp.where(s_hat == v1, row, big), axis=0, keepdims=True)
    masked = jnp.where(row == i1, jnp.float32(jnp.inf), s_hat)
    v2 = jnp.min(masked, axis=0, keepdims=True)
    i2 = jnp.min(jnp.where(masked == v2, row, big), axis=0, keepdims=True)

    i1c = jnp.transpose(i1.astype(jnp.float32)).astype(jnp.int32)
    i2c = jnp.transpose(i2.astype(jnp.float32)).astype(jnp.int32)

    x = jnp.transpose(xt_ref[:])
    col = jax.lax.broadcasted_iota(jnp.int32, (_B, _NCLS), 1)
    oh1 = (col == i1c).astype(jnp.float32)
    c1 = jax.lax.dot_general(oh1, cent, (((1,), (0,)), ((), ())),
                             precision=_HI)
    oh2 = (col == i2c).astype(jnp.float32)
    c2 = jax.lax.dot_general(oh2, cent, (((1,), (0,)), ((), ())),
                             precision=_HI)
    diff1 = c1 - x
    d1 = jnp.sum(diff1 * diff1, axis=-1, keepdims=True)
    diff2 = c2 - x
    d2 = jnp.sum(diff2 * diff2, axis=-1, keepdims=True)
    w = jnp.where(d1 < d2, i1c,
                  jnp.where(d2 < d1, i2c, jnp.minimum(i1c, i2c)))
    wt = jnp.transpose(w.astype(jnp.float32)).astype(jnp.int32)
    outt_ref[:] = jnp.where(row == wt, jnp.float32(_POS_INF), jnp.float32(0.0))


def kernel(x, attrs, z, G_W1, G_b1, G_W2, G_b2):
    outt = pl.pallas_call(
        _fgc_kernel,
        out_shape=jax.ShapeDtypeStruct((_NCLS, x.shape[0]), jnp.float32),
        compiler_params=pltpu.CompilerParams(vmem_limit_bytes=64 * 1024 * 1024),
    )(x.T, z.T, attrs.T, G_W1, G_b1.reshape(1, _HID), G_W2.T,
      G_b2.reshape(1, _XD))
    return outt.T


# speedup vs baseline: 2.4575x; 1.1649x over previous
import jax
import jax.numpy as jnp
from jax.experimental import pallas as pl
from jax.experimental.pallas import tpu as pltpu

_NCLS = 1000
_NS = 5
_HID = 512
_XD = 64
_B = 1024
_POS_INF = 1e6
_HI = jax.lax.Precision.HIGHEST


def _fgc_kernel(xt_ref, zt_ref, attrst_ref, w1_ref, b1_ref, w2t_ref, b2_ref,
                outt_ref):
    w1 = w1_ref[:]
    w2t = w2t_ref[:]
    b1 = b1_ref[:]
    attrst = attrst_ref[:]
    xfsum = jnp.zeros((_NCLS, _XD), jnp.float32)
    for s in range(_NS):
        z_st = zt_ref[:, pl.ds(s * _NCLS, _NCLS)]
        g_st = jnp.concatenate([z_st, attrst], axis=0)
        h_s = jnp.maximum(
            jax.lax.dot_general(g_st, w1, (((0,), (0,)), ((), ()))) + b1, 0.0)
        xfsum = xfsum + jax.lax.dot_general(
            h_s, w2t, (((1,), (1,)), ((), ())))
    cent = xfsum * jnp.float32(1.0 / _NS) + b2_ref[:]

    cn = jnp.sum(cent * cent, axis=1, keepdims=True)
    cx = jax.lax.dot_general(cent, xt_ref[:], (((1,), (0,)), ((), ())),
                             precision=_HI)
    s_hat = cn - (cx + cx)

    big = jnp.int32(2 ** 30)
    row = jax.lax.broadcasted_iota(jnp.int32, s_hat.shape, 0)
    v1 = jnp.min(s_hat, axis=0, keepdims=True)
    i1 = jnp.min(jnp.where(s_hat == v1, row, big), axis=0, keepdims=True)
    masked = jnp.where(row == i1, jnp.float32(jnp.inf), s_hat)
    v2 = jnp.min(masked, axis=0, keepdims=True)
    i2 = jnp.min(jnp.where(masked == v2, row, big), axis=0, keepdims=True)

    i1c = jnp.transpose(i1.astype(jnp.float32)).astype(jnp.int32)
    i2c = jnp.transpose(i2.astype(jnp.float32)).astype(jnp.int32)

    x = jnp.transpose(xt_ref[:])
    col = jax.lax.broadcasted_iota(jnp.int32, (_B, _NCLS), 1)
    p0 = cent.astype(jnp.bfloat16)
    r1 = cent - p0.astype(jnp.float32)
    p1 = r1.astype(jnp.bfloat16)
    p2 = (r1 - p1.astype(jnp.float32)).astype(jnp.bfloat16)
    cpack = jnp.concatenate([p0, p1, p2], axis=1)
    dnum = (((1,), (0,)), ((), ()))
    oh1 = (col == i1c).astype(jnp.bfloat16)
    g1 = jax.lax.dot_general(oh1, cpack, dnum,
                             preferred_element_type=jnp.float32)
    c1 = (g1[:, :_XD] + g1[:, _XD:2 * _XD]) + g1[:, 2 * _XD:]
    oh2 = (col == i2c).astype(jnp.bfloat16)
    g2 = jax.lax.dot_general(oh2, cpack, dnum,
                             preferred_element_type=jnp.float32)
    c2 = (g2[:, :_XD] + g2[:, _XD:2 * _XD]) + g2[:, 2 * _XD:]
    diff1 = c1 - x
    d1 = jnp.sum(diff1 * diff1, axis=-1, keepdims=True)
    diff2 = c2 - x
    d2 = jnp.sum(diff2 * diff2, axis=-1, keepdims=True)
    w = jnp.where(d1 < d2, i1c,
                  jnp.where(d2 < d1, i2c, jnp.minimum(i1c, i2c)))
    wt = jnp.transpose(w.astype(jnp.float32)).astype(jnp.int32)
    outt_ref[:] = jnp.where(row == wt, jnp.float32(_POS_INF), jnp.float32(0.0))


def kernel(x, attrs, z, G_W1, G_b1, G_W2, G_b2):
    outt = pl.pallas_call(
        _fgc_kernel,
        out_shape=jax.ShapeDtypeStruct((_NCLS, x.shape[0]), jnp.float32),
        compiler_params=pltpu.CompilerParams(vmem_limit_bytes=64 * 1024 * 1024),
    )(x.T, z.T, attrs.T, G_W1, G_b1.reshape(1, _HID), G_W2.T,
      G_b2.reshape(1, _XD))
    return outt.T
